# baseline (speedup 1.0000x reference)
import functools

import jax
import jax.numpy as jnp
from jax import lax
from jax.experimental import pallas as pl
from jax.experimental.pallas import tpu as pltpu
from jax.experimental.pallas import tpu_sc as plsc

N = 10000
E = 320000
HID = 128
EF = 16
NG = 16
XP = 16
CUTOFF = 10.0

NC = 2
NS = 16
NW = NC * NS

K = 5
EC = E // K
EWC = EC // NW
CH = 80
NIT = EWC // CH
NPT = N // NS

BE = 3200
BN = 1000


def _pre_body(h_ref, x_ref, wa_ref, wb_ref, ha_ref, hb_ref, xp_ref):
    h = h_ref[...]
    ha_ref[...] = jnp.dot(
        h, wa_ref[...], preferred_element_type=jnp.float32).astype(jnp.bfloat16)
    hb_ref[...] = jnp.dot(
        h, wb_ref[...], preferred_element_type=jnp.float32).astype(jnp.bfloat16)
    x = x_ref[...]
    xp_ref[...] = jnp.concatenate(
        [x, jnp.zeros((x.shape[0], XP - 3), jnp.float32)], axis=1)


def _pre_call(h, x, wa, wb):
    nb = N // BN
    return pl.pallas_call(
        _pre_body,
        grid=(nb,),
        in_specs=[
            pl.BlockSpec((BN, HID), lambda i: (i, 0)),
            pl.BlockSpec((BN, 3), lambda i: (i, 0)),
            pl.BlockSpec((HID, HID), lambda i: (0, 0)),
            pl.BlockSpec((HID, HID), lambda i: (0, 0)),
        ],
        out_specs=[
            pl.BlockSpec((BN, HID), lambda i: (i, 0)),
            pl.BlockSpec((BN, HID), lambda i: (i, 0)),
            pl.BlockSpec((BN, XP), lambda i: (i, 0)),
        ],
        out_shape=[
            jax.ShapeDtypeStruct((N, HID), jnp.bfloat16),
            jax.ShapeDtypeStruct((N, HID), jnp.bfloat16),
            jax.ShapeDtypeStruct((N, XP), jnp.float32),
        ],
    )(h, x, wa, wb)


def _gather_body(ha_h, hb_h, xp_h, dst_h, src_h,
                 pa_h, pb_h, xd_h, xs_h,
                 idxd0, idxs0, ra0, rb0, xd0, xs0,
                 idxd1, idxs1, ra1, rb1, xd1, xs1,
                 semI, semG, semO0, semO1):
    wid = lax.axis_index("s") * NC + lax.axis_index("c")
    base = wid * EWC
    bufs = ((idxd0, idxs0, ra0, rb0, xd0, xs0, semO0),
            (idxd1, idxs1, ra1, rb1, xd1, xs1, semO1))

    def start_idx(j, b):
        idxd, idxs = bufs[b][0], bufs[b][1]
        off = base + j * CH
        pltpu.async_copy(dst_h.at[pl.ds(off, CH)], idxd, semI)
        pltpu.async_copy(src_h.at[pl.ds(off, CH)], idxs, semI)

    def wait_idx(b):
        idxd, idxs = bufs[b][0], bufs[b][1]
        pltpu.make_async_copy(dst_h.at[pl.ds(0, CH)], idxd, semI).wait()
        pltpu.make_async_copy(src_h.at[pl.ds(0, CH)], idxs, semI).wait()

    def gathers(b):
        idxd, idxs, ra, rb, xdv, xsv = bufs[b][:6]
        c1 = pltpu.async_copy(ha_h.at[idxd], ra, semG)
        c2 = pltpu.async_copy(hb_h.at[idxs], rb, semG)
        c3 = pltpu.async_copy(xp_h.at[idxd], xdv, semG)
        c4 = pltpu.async_copy(xp_h.at[idxs], xsv, semG)
        c1.wait()
        c2.wait()
        c3.wait()
        c4.wait()

    def start_out(j, b):
        _, _, ra, rb, xdv, xsv, semO = bufs[b]
        off = base + j * CH
        qq = off % BE
        row = 400 * (off // BE) + qq % 400
        lane = 16 * (qq // 400)
        pltpu.async_copy(ra, pa_h.at[pl.ds(off, CH)], semO)
        pltpu.async_copy(rb, pb_h.at[pl.ds(off, CH)], semO)
        pltpu.async_copy(xdv, xd_h.at[pl.ds(row, CH), pl.ds(lane, XP)], semO)
        pltpu.async_copy(xsv, xs_h.at[pl.ds(row, CH), pl.ds(lane, XP)], semO)

    def drain_out(b):
        _, _, ra, rb, xdv, xsv, semO = bufs[b]
        pltpu.make_async_copy(ra, pa_h.at[pl.ds(0, CH)], semO).wait()
        pltpu.make_async_copy(rb, pb_h.at[pl.ds(0, CH)], semO).wait()
        pltpu.make_async_copy(
            xdv, xd_h.at[pl.ds(0, CH), pl.ds(0, XP)], semO).wait()
        pltpu.make_async_copy(
            xsv, xs_h.at[pl.ds(0, CH), pl.ds(0, XP)], semO).wait()

    def round_(j, b, drain):
        wait_idx(b)
        start_idx(j + 1, 1 - b)
        if drain:
            drain_out(b)
        gathers(b)
        start_out(j, b)

    start_idx(0, 0)
    wait_idx(0)
    start_idx(1, 1)
    gathers(0)
    start_out(0, 0)
    round_(1, 1, False)

    def body(jj, carry):
        round_(2 + 2 * jj, 0, True)
        round_(3 + 2 * jj, 1, True)
        return carry

    lax.fori_loop(0, (NIT - 2) // 2, body, 0)

    wait_idx(0)
    drain_out(0)
    gathers(0)
    start_out(NIT - 1, 0)
    drain_out(1)
    drain_out(0)


def _gather_call(ha, hb, xp, dst, src):
    mesh = plsc.VectorSubcoreMesh(core_axis_name="c", subcore_axis_name="s")
    f = pl.kernel(
        _gather_body,
        out_type=(
            jax.ShapeDtypeStruct((EC, HID), jnp.bfloat16),
            jax.ShapeDtypeStruct((EC, HID), jnp.bfloat16),
            jax.ShapeDtypeStruct((EC // 8, 128), jnp.float32),
            jax.ShapeDtypeStruct((EC // 8, 128), jnp.float32),
        ),
        mesh=mesh,
        scratch_types=[
            pltpu.VMEM((CH,), jnp.int32),
            pltpu.VMEM((CH,), jnp.int32),
            pltpu.VMEM((CH, HID), jnp.bfloat16),
            pltpu.VMEM((CH, HID), jnp.bfloat16),
            pltpu.VMEM((CH, XP), jnp.float32),
            pltpu.VMEM((CH, XP), jnp.float32),
            pltpu.VMEM((CH,), jnp.int32),
            pltpu.VMEM((CH,), jnp.int32),
            pltpu.VMEM((CH, HID), jnp.bfloat16),
            pltpu.VMEM((CH, HID), jnp.bfloat16),
            pltpu.VMEM((CH, XP), jnp.float32),
            pltpu.VMEM((CH, XP), jnp.float32),
            pltpu.SemaphoreType.DMA,
            pltpu.SemaphoreType.DMA,
            pltpu.SemaphoreType.DMA,
            pltpu.SemaphoreType.DMA,
        ],
        compiler_params=pltpu.CompilerParams(use_tc_tiling_on_sc=False),
    )
    return f(ha, hb, xp, dst, src)


def _edge_body(pa_ref, pb_ref, xd_ref, xs_ref, ea_ref,
               b1_ref, w1e_ref, w2_ref, b2_ref,
               gw_ref, gb_ref, xw1_ref, xb1_ref, xw2_ref,
               mg_ref, wv_ref):
    SB = BE // 8
    relp = xd_ref[...] - xs_ref[...]
    eap = ea_ref[...]
    step = CUTOFF / (NG - 1)
    offset = lax.broadcasted_iota(jnp.int32, (1, NG), 1).astype(jnp.float32) * step
    coeff = -0.5 / (step * step)
    wv_parts = []
    for j in range(8):
        rel = relp[:, 16 * j:16 * j + XP]
        d2 = jnp.sum(rel * rel, axis=1, keepdims=True)
        r = jnp.sqrt(d2 + 1e-8)
        d_feat = jnp.exp(coeff * (r - offset) ** 2)
        ef = jnp.concatenate([d_feat, eap[:, 16 * j:16 * j + EF]], axis=1)
        t0 = (pa_ref[pl.ds(SB * j, SB), :].astype(jnp.float32)
              + pb_ref[pl.ds(SB * j, SB), :].astype(jnp.float32)
              + b1_ref[...] + jnp.dot(ef, w1e_ref[...],
                                      preferred_element_type=jnp.float32))
        t1 = jax.nn.silu(t0)
        m = jax.nn.silu(jnp.dot(t1, w2_ref[...],
                                preferred_element_type=jnp.float32) + b2_ref[...])
        g = jax.nn.sigmoid(jnp.dot(m, gw_ref[...],
                                   preferred_element_type=jnp.float32) + gb_ref[...])
        mg_ref[pl.ds(SB * j, SB), :] = m * g
        c1 = jax.nn.silu(jnp.dot(m, xw1_ref[...],
                                 preferred_element_type=jnp.float32) + xb1_ref[...])
        coef = jnp.tanh(jnp.dot(c1, xw2_ref[...],
                                preferred_element_type=jnp.float32))
        wv_parts.append(rel * (coef / (r + 1.0)))
    wv_ref[...] = jnp.concatenate(wv_parts, axis=1)


def _edge_call(pa, pb, xd, xs, ea, b1, w1e, w2, b2, gw, gb, xw1, xb1, xw2):
    nb = EC // BE
    full = lambda s: pl.BlockSpec(s, lambda i: (0, 0))
    blk = lambda w: pl.BlockSpec((BE, w), lambda i: (i, 0))
    pk = lambda: pl.BlockSpec((BE // 8, 128), lambda i: (i, 0))
    return pl.pallas_call(
        _edge_body,
        grid=(nb,),
        in_specs=[
            blk(HID), blk(HID), pk(), pk(), pk(),
            full((1, HID)), full((2 * NG, HID)), full((HID, HID)),
            full((1, HID)), full((HID, 1)), full((1, 1)),
            full((HID, HID)), full((1, HID)), full((HID, 1)),
        ],
        out_specs=[blk(HID), pk()],
        out_shape=[
            jax.ShapeDtypeStruct((EC, HID), jnp.float32),
            jax.ShapeDtypeStruct((EC // 8, 128), jnp.float32),
        ],
    )(pa, pb, xd, xs, ea, b1, w1e, w2, b2, gw, gb, xw1, xb1, xw2)


def _scatter_body(nch, *refs):
    chunk_refs = [refs[3 * k:3 * k + 3] for k in range(nch)]
    (z128_h, z16_h, agg_h, xagg_h,
     idxd0, rows0, wrows0, idxd1, rows1, wrows1,
     hacc, xacc, semL, semS0, semS1) = refs[3 * nch:]
    cid = lax.axis_index("c")
    sid = lax.axis_index("s")
    wid = sid * NC + cid
    base = wid * EWC
    bufs = ((idxd0, rows0, wrows0, semS0), (idxd1, rows1, wrows1, semS1))

    pltpu.sync_copy(z128_h, hacc.at[pl.ds(sid * NPT, NPT)])
    pltpu.sync_copy(z16_h, xacc.at[pl.ds(sid * NPT, NPT)])
    plsc.subcore_barrier()

    def chunk_phase(mg_h, wv_h, dst_h):
        def start_loads(j, b):
            idxd, rows, wrows, _ = bufs[b]
            off = base + j * CH
            qq = off % BE
            row = 400 * (off // BE) + qq % 400
            lane = 16 * (qq // 400)
            pltpu.async_copy(dst_h.at[pl.ds(off, CH)], idxd, semL)
            pltpu.async_copy(mg_h.at[pl.ds(off, CH)], rows, semL)
            pltpu.async_copy(wv_h.at[pl.ds(row, CH), pl.ds(lane, XP)], wrows,
                             semL)

        def wait_loads(b):
            idxd, rows, wrows, _ = bufs[b]
            pltpu.make_async_copy(dst_h.at[pl.ds(0, CH)], idxd, semL).wait()
            pltpu.make_async_copy(mg_h.at[pl.ds(0, CH)], rows, semL).wait()
            pltpu.make_async_copy(
                wv_h.at[pl.ds(0, CH), pl.ds(0, XP)], wrows, semL).wait()

        def start_scat(b):
            idxd, rows, wrows, semS = bufs[b]
            pltpu.async_copy(rows, hacc.at[idxd], semS, add=True)
            pltpu.async_copy(wrows, xacc.at[idxd], semS, add=True)

        def drain_scat(b):
            idxd, rows, wrows, semS = bufs[b]
            pltpu.make_async_copy(rows, hacc.at[idxd], semS).wait()
            pltpu.make_async_copy(wrows, xacc.at[idxd], semS).wait()

        start_loads(0, 0)
        wait_loads(0)
        start_loads(1, 1)
        start_scat(0)

        def round_(j, b):
            wait_loads(b)
            drain_scat(1 - b)
            start_loads(j + 1, 1 - b)
            start_scat(b)

        def body(jj, carry):
            round_(1 + 2 * jj, 1)
            round_(2 + 2 * jj, 0)
            return carry

        lax.fori_loop(0, (NIT - 3) // 2, body, 0)

        wait_loads(1)
        drain_scat(0)
        start_loads(NIT - 1, 0)
        start_scat(1)
        wait_loads(0)
        drain_scat(1)
        start_scat(0)
        drain_scat(0)

    for mg_h, wv_h, dst_h in chunk_refs:
        chunk_phase(mg_h, wv_h, dst_h)
    plsc.subcore_barrier()

    pltpu.sync_copy(hacc.at[pl.ds(sid * NPT, NPT)],
                    agg_h.at[cid, pl.ds(sid * NPT, NPT)])
    pltpu.sync_copy(xacc.at[pl.ds(sid * NPT, NPT)],
                    xagg_h.at[cid, pl.ds(sid * NPT, NPT)])


def _scatter_call(chunks, z128, z16):
    mesh = plsc.VectorSubcoreMesh(core_axis_name="c", subcore_axis_name="s")
    f = pl.kernel(
        functools.partial(_scatter_body, len(chunks)),
        out_type=(
            jax.ShapeDtypeStruct((NC, N, HID), jnp.float32),
            jax.ShapeDtypeStruct((NC, N, XP), jnp.float32),
        ),
        mesh=mesh,
        scratch_types=[
            pltpu.VMEM((CH,), jnp.int32),
            pltpu.VMEM((CH, HID), jnp.float32),
            pltpu.VMEM((CH, XP), jnp.float32),
            pltpu.VMEM((CH,), jnp.int32),
            pltpu.VMEM((CH, HID), jnp.float32),
            pltpu.VMEM((CH, XP), jnp.float32),
            pltpu.VMEM_SHARED((N, HID), jnp.float32),
            pltpu.VMEM_SHARED((N, XP), jnp.float32),
            pltpu.SemaphoreType.DMA,
            pltpu.SemaphoreType.DMA,
            pltpu.SemaphoreType.DMA,
        ],
        compiler_params=pltpu.CompilerParams(use_tc_tiling_on_sc=False),
    )
    flat = [r for ch in chunks for r in ch]
    return f(*flat, z128, z16)


def _node_body(ns, *refs):
    (h_ref, x_ref), a_refs = refs[:2], refs[2:2 + ns]
    xa_refs = refs[2 + ns:2 + 2 * ns]
    (mk_ref, w1a_ref, w1b_ref, b1_ref, w2_ref, b2_ref,
     hn_ref, xn_ref) = refs[2 + 2 * ns:]
    h = h_ref[...]
    agg = sum(a[0] + a[1] for a in a_refs)
    u = jax.nn.silu(
        jnp.dot(agg, w1a_ref[...], preferred_element_type=jnp.float32)
        + jnp.dot(h, w1b_ref[...], preferred_element_type=jnp.float32)
        + b1_ref[...])
    hn_ref[...] = h + jnp.dot(u, w2_ref[...],
                              preferred_element_type=jnp.float32) + b2_ref[...]
    xagg = sum(xa[0] + xa[1] for xa in xa_refs)
    xn_ref[...] = x_ref[...] + xagg[:, :3] * mk_ref[...]


def _node_call(h, x, aggs, xaggs, mk, w1a, w1b, b1, w2, b2):
    nb = N // BN
    ns = len(aggs)
    full = lambda s: pl.BlockSpec(s, lambda i: (0, 0))
    blk = lambda w: pl.BlockSpec((BN, w), lambda i: (i, 0))
    blk3 = lambda w: pl.BlockSpec((NC, BN, w), lambda i: (0, i, 0))
    return pl.pallas_call(
        functools.partial(_node_body, ns),
        grid=(nb,),
        in_specs=[
            blk(HID), blk(3),
            *([blk3(HID)] * ns), *([blk3(XP)] * ns),
            blk(1),
            full((HID, HID)), full((HID, HID)), full((1, HID)),
            full((HID, HID)), full((1, HID)),
        ],
        out_specs=[blk(HID), blk(3)],
        out_shape=[
            jax.ShapeDtypeStruct((N, HID), jnp.float32),
            jax.ShapeDtypeStruct((N, 3), jnp.float32),
        ],
    )(h, x, *aggs, *xaggs, mk, w1a, w1b, b1, w2, b2)


def kernel(h, x, edge_index, mask_ligand, edge_attr,
           edge_W1, edge_b1, edge_W2, edge_b2,
           gate_W, gate_b,
           node_W1, node_b1, node_W2, node_b2,
           xm_W1, xm_b1, xm_W2):
    ei = edge_index.astype(jnp.int32)
    src = ei[0]
    dst = ei[1]

    ha, hb, xp = _pre_call(h, x, edge_W1[:HID], edge_W1[HID:2 * HID])

    eap = jnp.reshape(
        jnp.transpose(jnp.reshape(edge_attr, (E // BE, 8, BE // 8, EF)),
                      (0, 2, 1, 3)),
        (E // 8, 128))

    dsts, mgs, wvs = [], [], []
    for k in range(K):
        dk = lax.slice(dst, (k * EC,), ((k + 1) * EC,))
        sk = lax.slice(src, (k * EC,), ((k + 1) * EC,))
        pa, pb, xd, xs = _gather_call(ha, hb, xp, dk, sk)
        eak = lax.slice(eap, (k * (EC // 8), 0), ((k + 1) * (EC // 8), 128))
        mg, wv = _edge_call(
            pa, pb, xd, xs, eak,
            edge_b1.reshape(1, HID), edge_W1[2 * HID:], edge_W2,
            edge_b2.reshape(1, HID), gate_W, gate_b.reshape(1, 1),
            xm_W1, xm_b1.reshape(1, HID), xm_W2)
        dsts.append(dk)
        mgs.append(mg)
        wvs.append(wv)

    z128 = jnp.zeros((NPT, HID), jnp.float32)
    z16 = jnp.zeros((NPT, XP), jnp.float32)
    aggs, xaggs = [], []
    for group in ((0, 1), (2, 3), (4,)):
        a, xa = _scatter_call(
            [(mgs[k], wvs[k], dsts[k]) for k in group], z128, z16)
        aggs.append(a)
        xaggs.append(xa)

    mk = mask_ligand.astype(jnp.float32).reshape(N, 1)
    h_new, x_new = _node_call(
        h, x, aggs, xaggs, mk,
        node_W1[:HID], node_W1[HID:], node_b1.reshape(1, HID),
        node_W2, node_b2.reshape(1, HID))
    return (h_new, x_new)


# speedup vs baseline: 1.9363x; 1.9363x over previous
import functools

import jax
import jax.numpy as jnp
from jax import lax
from jax.experimental import pallas as pl
from jax.experimental.pallas import tpu as pltpu
from jax.experimental.pallas import tpu_sc as plsc

N = 10000
E = 320000
HID = 128
EF = 16
NG = 16
XP = 16
CUTOFF = 10.0

NC = 2
NS = 16
NW = NC * NS

K = 5
EC = E // K
EWC = EC // NW
CH = 80
NIT = EWC // CH
NPT = N // NS

BE = 3200
BN = 1000


def _pack2(z):
    bits = lax.bitcast_convert_type(z, jnp.uint32)
    lo = bits[:, :64]
    hi = bits[:, 64:]
    pk = (((lo + jnp.uint32(0x8000)) >> jnp.uint32(16))
          | ((hi + jnp.uint32(0x8000)) & jnp.uint32(0xFFFF0000)))
    return lax.bitcast_convert_type(pk, jnp.float32)


def _pre_body(h_ref, x_ref, wa_ref, wb_ref, ha_ref, hb_ref, xp_ref):
    h = h_ref[...]
    za = jnp.dot(h, wa_ref[...], preferred_element_type=jnp.float32)
    zb = jnp.dot(h, wb_ref[...], preferred_element_type=jnp.float32)
    ha_ref[...] = _pack2(za)
    hb_ref[...] = _pack2(zb)
    x = x_ref[...]
    xp_ref[...] = jnp.concatenate(
        [x, jnp.zeros((x.shape[0], XP - 3), jnp.float32)], axis=1)


def _pre_call(h, x, wa, wb):
    nb = N // BN
    return pl.pallas_call(
        _pre_body,
        grid=(nb,),
        in_specs=[
            pl.BlockSpec((BN, HID), lambda i: (i, 0)),
            pl.BlockSpec((BN, 3), lambda i: (i, 0)),
            pl.BlockSpec((HID, HID), lambda i: (0, 0)),
            pl.BlockSpec((HID, HID), lambda i: (0, 0)),
        ],
        out_specs=[
            pl.BlockSpec((BN, 64), lambda i: (i, 0)),
            pl.BlockSpec((BN, 64), lambda i: (i, 0)),
            pl.BlockSpec((BN, XP), lambda i: (i, 0)),
        ],
        out_shape=[
            jax.ShapeDtypeStruct((N, 64), jnp.float32),
            jax.ShapeDtypeStruct((N, 64), jnp.float32),
            jax.ShapeDtypeStruct((N, XP), jnp.float32),
        ],
    )(h, x, wa, wb)


def _gather_body(ha_h, hb_h, xp_h, dst_h, src_h,
                 pab_h, xd_h, xs_h,
                 idxd0, idxs0, ra0, rb0, xd0, xs0,
                 idxd1, idxs1, ra1, rb1, xd1, xs1,
                 semI, semG, semO0, semO1):
    wid = lax.axis_index("s") * NC + lax.axis_index("c")
    base = wid * EWC
    bufs = ((idxd0, idxs0, ra0, rb0, xd0, xs0, semO0),
            (idxd1, idxs1, ra1, rb1, xd1, xs1, semO1))

    def start_idx(j, b):
        idxd, idxs = bufs[b][0], bufs[b][1]
        off = base + j * CH
        pltpu.async_copy(dst_h.at[pl.ds(off, CH)], idxd, semI)
        pltpu.async_copy(src_h.at[pl.ds(off, CH)], idxs, semI)

    def wait_idx(b):
        idxd, idxs = bufs[b][0], bufs[b][1]
        pltpu.make_async_copy(dst_h.at[pl.ds(0, CH)], idxd, semI).wait()
        pltpu.make_async_copy(src_h.at[pl.ds(0, CH)], idxs, semI).wait()

    def gathers(b):
        idxd, idxs, ra, rb, xdv, xsv = bufs[b][:6]
        c1 = pltpu.async_copy(ha_h.at[idxd], ra, semG)
        c2 = pltpu.async_copy(hb_h.at[idxs], rb, semG)
        c3 = pltpu.async_copy(xp_h.at[idxd], xdv, semG)
        c4 = pltpu.async_copy(xp_h.at[idxs], xsv, semG)
        c1.wait()
        c2.wait()
        c3.wait()
        c4.wait()

    def start_out(j, b):
        _, _, ra, rb, xdv, xsv, semO = bufs[b]
        off = base + j * CH
        qq = off % BE
        row = 400 * (off // BE) + qq % 400
        lane = 16 * (qq // 400)
        pltpu.async_copy(ra, pab_h.at[pl.ds(off, CH), pl.ds(0, 64)], semO)
        pltpu.async_copy(rb, pab_h.at[pl.ds(off, CH), pl.ds(64, 64)], semO)
        pltpu.async_copy(xdv, xd_h.at[pl.ds(row, CH), pl.ds(lane, XP)], semO)
        pltpu.async_copy(xsv, xs_h.at[pl.ds(row, CH), pl.ds(lane, XP)], semO)

    def drain_out(b):
        _, _, ra, rb, xdv, xsv, semO = bufs[b]
        pltpu.make_async_copy(
            ra, pab_h.at[pl.ds(0, CH), pl.ds(0, 64)], semO).wait()
        pltpu.make_async_copy(
            rb, pab_h.at[pl.ds(0, CH), pl.ds(0, 64)], semO).wait()
        pltpu.make_async_copy(
            xdv, xd_h.at[pl.ds(0, CH), pl.ds(0, XP)], semO).wait()
        pltpu.make_async_copy(
            xsv, xs_h.at[pl.ds(0, CH), pl.ds(0, XP)], semO).wait()

    def round_(j, b, drain):
        wait_idx(b)
        start_idx(j + 1, 1 - b)
        if drain:
            drain_out(b)
        gathers(b)
        start_out(j, b)

    start_idx(0, 0)
    wait_idx(0)
    start_idx(1, 1)
    gathers(0)
    start_out(0, 0)
    round_(1, 1, False)

    def body(jj, carry):
        round_(2 + 2 * jj, 0, True)
        round_(3 + 2 * jj, 1, True)
        return carry

    lax.fori_loop(0, (NIT - 2) // 2, body, 0)

    wait_idx(0)
    drain_out(0)
    gathers(0)
    start_out(NIT - 1, 0)
    drain_out(1)
    drain_out(0)


def _gather_call(ha, hb, xp, dst, src):
    mesh = plsc.VectorSubcoreMesh(core_axis_name="c", subcore_axis_name="s")
    f = pl.kernel(
        _gather_body,
        out_type=(
            jax.ShapeDtypeStruct((EC, HID), jnp.float32),
            jax.ShapeDtypeStruct((EC // 8, 128), jnp.float32),
            jax.ShapeDtypeStruct((EC // 8, 128), jnp.float32),
        ),
        mesh=mesh,
        scratch_types=[
            pltpu.VMEM((CH,), jnp.int32),
            pltpu.VMEM((CH,), jnp.int32),
            pltpu.VMEM((CH, 64), jnp.float32),
            pltpu.VMEM((CH, 64), jnp.float32),
            pltpu.VMEM((CH, XP), jnp.float32),
            pltpu.VMEM((CH, XP), jnp.float32),
            pltpu.VMEM((CH,), jnp.int32),
            pltpu.VMEM((CH,), jnp.int32),
            pltpu.VMEM((CH, 64), jnp.float32),
            pltpu.VMEM((CH, 64), jnp.float32),
            pltpu.VMEM((CH, XP), jnp.float32),
            pltpu.VMEM((CH, XP), jnp.float32),
            pltpu.SemaphoreType.DMA,
            pltpu.SemaphoreType.DMA,
            pltpu.SemaphoreType.DMA,
            pltpu.SemaphoreType.DMA,
        ],
        compiler_params=pltpu.CompilerParams(use_tc_tiling_on_sc=False),
    )
    return f(ha, hb, xp, dst, src)


def _edge_body(pab_ref, xd_ref, xs_ref, ea_ref,
               b1_ref, w1e_ref, w2_ref, b2_ref,
               gw_ref, gb_ref, xw1_ref, xb1_ref, xw2_ref,
               mg_ref, wv_ref):
    SB = BE // 8
    relp = xd_ref[...] - xs_ref[...]
    eap = ea_ref[...]
    step = CUTOFF / (NG - 1)
    offset = lax.broadcasted_iota(jnp.int32, (1, NG), 1).astype(jnp.float32) * step
    coeff = -0.5 / (step * step)
    wv_parts = []
    for j in range(8):
        rel = relp[:, 16 * j:16 * j + XP]
        d2 = jnp.sum(rel * rel, axis=1, keepdims=True)
        r = jnp.sqrt(d2 + 1e-8)
        d_feat = jnp.exp(coeff * (r - offset) ** 2)
        ef = jnp.concatenate([d_feat, eap[:, 16 * j:16 * j + EF]], axis=1)
        bits = lax.bitcast_convert_type(pab_ref[pl.ds(SB * j, SB), :],
                                        jnp.uint32)
        ba = bits[:, :64]
        bb = bits[:, 64:]
        ev = lax.bitcast_convert_type((ba << jnp.uint32(16)), jnp.float32) \
            + lax.bitcast_convert_type((bb << jnp.uint32(16)), jnp.float32)
        od = lax.bitcast_convert_type(ba & jnp.uint32(0xFFFF0000), jnp.float32) \
            + lax.bitcast_convert_type(bb & jnp.uint32(0xFFFF0000), jnp.float32)
        t0 = (jnp.concatenate([ev, od], axis=1)
              + b1_ref[...] + jnp.dot(ef, w1e_ref[...],
                                      preferred_element_type=jnp.float32))
        t1 = jax.nn.silu(t0)
        m = jax.nn.silu(jnp.dot(t1, w2_ref[...],
                                preferred_element_type=jnp.float32) + b2_ref[...])
        g = jax.nn.sigmoid(jnp.dot(m, gw_ref[...],
                                   preferred_element_type=jnp.float32) + gb_ref[...])
        mg_ref[pl.ds(SB * j, SB), :] = m * g
        c1 = jax.nn.silu(jnp.dot(m, xw1_ref[...],
                                 preferred_element_type=jnp.float32) + xb1_ref[...])
        coef = jnp.tanh(jnp.dot(c1, xw2_ref[...],
                                preferred_element_type=jnp.float32))
        wv_parts.append(rel * (coef / (r + 1.0)))
    wv_ref[...] = jnp.concatenate(wv_parts, axis=1)


def _edge_call(pab, xd, xs, ea, b1, w1e, w2, b2, gw, gb, xw1, xb1, xw2):
    nb = EC // BE
    full = lambda s: pl.BlockSpec(s, lambda i: (0, 0))
    blk = lambda w: pl.BlockSpec((BE, w), lambda i: (i, 0))
    pk = lambda: pl.BlockSpec((BE // 8, 128), lambda i: (i, 0))
    return pl.pallas_call(
        _edge_body,
        grid=(nb,),
        in_specs=[
            blk(HID), pk(), pk(), pk(),
            full((1, HID)), full((2 * NG, HID)), full((HID, HID)),
            full((1, HID)), full((HID, 1)), full((1, 1)),
            full((HID, HID)), full((1, HID)), full((HID, 1)),
        ],
        out_specs=[blk(HID), pk()],
        out_shape=[
            jax.ShapeDtypeStruct((EC, HID), jnp.float32),
            jax.ShapeDtypeStruct((EC // 8, 128), jnp.float32),
        ],
    )(pab, xd, xs, ea, b1, w1e, w2, b2, gw, gb, xw1, xb1, xw2)


def _scatter_body(nch, *refs):
    chunk_refs = [refs[3 * k:3 * k + 3] for k in range(nch)]
    (z128_h, z16_h, agg_h, xagg_h,
     idxd0, rows0, wrows0, idxd1, rows1, wrows1,
     hacc, xacc, semL, semS0, semS1) = refs[3 * nch:]
    cid = lax.axis_index("c")
    sid = lax.axis_index("s")
    wid = sid * NC + cid
    base = wid * EWC
    bufs = ((idxd0, rows0, wrows0, semS0), (idxd1, rows1, wrows1, semS1))

    pltpu.sync_copy(z128_h, hacc.at[pl.ds(sid * NPT, NPT)])
    pltpu.sync_copy(z16_h, xacc.at[pl.ds(sid * NPT, NPT)])
    plsc.subcore_barrier()

    def chunk_phase(mg_h, wv_h, dst_h):
        def start_loads(j, b):
            idxd, rows, wrows, _ = bufs[b]
            off = base + j * CH
            qq = off % BE
            row = 400 * (off // BE) + qq % 400
            lane = 16 * (qq // 400)
            pltpu.async_copy(dst_h.at[pl.ds(off, CH)], idxd, semL)
            pltpu.async_copy(mg_h.at[pl.ds(off, CH)], rows, semL)
            pltpu.async_copy(wv_h.at[pl.ds(row, CH), pl.ds(lane, XP)], wrows,
                             semL)

        def wait_loads(b):
            idxd, rows, wrows, _ = bufs[b]
            pltpu.make_async_copy(dst_h.at[pl.ds(0, CH)], idxd, semL).wait()
            pltpu.make_async_copy(mg_h.at[pl.ds(0, CH)], rows, semL).wait()
            pltpu.make_async_copy(
                wv_h.at[pl.ds(0, CH), pl.ds(0, XP)], wrows, semL).wait()

        def start_scat(b):
            idxd, rows, wrows, semS = bufs[b]
            pltpu.async_copy(rows, hacc.at[idxd], semS, add=True)
            pltpu.async_copy(wrows, xacc.at[idxd], semS, add=True)

        def drain_scat(b):
            idxd, rows, wrows, semS = bufs[b]
            pltpu.make_async_copy(rows, hacc.at[idxd], semS).wait()
            pltpu.make_async_copy(wrows, xacc.at[idxd], semS).wait()

        start_loads(0, 0)
        wait_loads(0)
        start_loads(1, 1)
        start_scat(0)

        def round_(j, b):
            wait_loads(b)
            drain_scat(1 - b)
            start_loads(j + 1, 1 - b)
            start_scat(b)

        def body(jj, carry):
            round_(1 + 2 * jj, 1)
            round_(2 + 2 * jj, 0)
            return carry

        lax.fori_loop(0, (NIT - 3) // 2, body, 0)

        wait_loads(1)
        drain_scat(0)
        start_loads(NIT - 1, 0)
        start_scat(1)
        wait_loads(0)
        drain_scat(1)
        start_scat(0)
        drain_scat(0)

    for mg_h, wv_h, dst_h in chunk_refs:
        chunk_phase(mg_h, wv_h, dst_h)
    plsc.subcore_barrier()

    pltpu.sync_copy(hacc.at[pl.ds(sid * NPT, NPT)],
                    agg_h.at[cid, pl.ds(sid * NPT, NPT)])
    pltpu.sync_copy(xacc.at[pl.ds(sid * NPT, NPT)],
                    xagg_h.at[cid, pl.ds(sid * NPT, NPT)])


def _scatter_call(chunks, z128, z16):
    mesh = plsc.VectorSubcoreMesh(core_axis_name="c", subcore_axis_name="s")
    f = pl.kernel(
        functools.partial(_scatter_body, len(chunks)),
        out_type=(
            jax.ShapeDtypeStruct((NC, N, HID), jnp.float32),
            jax.ShapeDtypeStruct((NC, N, XP), jnp.float32),
        ),
        mesh=mesh,
        scratch_types=[
            pltpu.VMEM((CH,), jnp.int32),
            pltpu.VMEM((CH, HID), jnp.float32),
            pltpu.VMEM((CH, XP), jnp.float32),
            pltpu.VMEM((CH,), jnp.int32),
            pltpu.VMEM((CH, HID), jnp.float32),
            pltpu.VMEM((CH, XP), jnp.float32),
            pltpu.VMEM_SHARED((N, HID), jnp.float32),
            pltpu.VMEM_SHARED((N, XP), jnp.float32),
            pltpu.SemaphoreType.DMA,
            pltpu.SemaphoreType.DMA,
            pltpu.SemaphoreType.DMA,
        ],
        compiler_params=pltpu.CompilerParams(use_tc_tiling_on_sc=False),
    )
    flat = [r for ch in chunks for r in ch]
    return f(*flat, z128, z16)


def _node_body(ns, *refs):
    (h_ref, x_ref), a_refs = refs[:2], refs[2:2 + ns]
    xa_refs = refs[2 + ns:2 + 2 * ns]
    (mk_ref, w1a_ref, w1b_ref, b1_ref, w2_ref, b2_ref,
     hn_ref, xn_ref) = refs[2 + 2 * ns:]
    h = h_ref[...]
    agg = sum(a[0] + a[1] for a in a_refs)
    u = jax.nn.silu(
        jnp.dot(agg, w1a_ref[...], preferred_element_type=jnp.float32)
        + jnp.dot(h, w1b_ref[...], preferred_element_type=jnp.float32)
        + b1_ref[...])
    hn_ref[...] = h + jnp.dot(u, w2_ref[...],
                              preferred_element_type=jnp.float32) + b2_ref[...]
    xagg = sum(xa[0] + xa[1] for xa in xa_refs)
    xn_ref[...] = x_ref[...] + xagg[:, :3] * mk_ref[...]


def _node_call(h, x, aggs, xaggs, mk, w1a, w1b, b1, w2, b2):
    nb = N // BN
    ns = len(aggs)
    full = lambda s: pl.BlockSpec(s, lambda i: (0, 0))
    blk = lambda w: pl.BlockSpec((BN, w), lambda i: (i, 0))
    blk3 = lambda w: pl.BlockSpec((NC, BN, w), lambda i: (0, i, 0))
    return pl.pallas_call(
        functools.partial(_node_body, ns),
        grid=(nb,),
        in_specs=[
            blk(HID), blk(3),
            *([blk3(HID)] * ns), *([blk3(XP)] * ns),
            blk(1),
            full((HID, HID)), full((HID, HID)), full((1, HID)),
            full((HID, HID)), full((1, HID)),
        ],
        out_specs=[blk(HID), blk(3)],
        out_shape=[
            jax.ShapeDtypeStruct((N, HID), jnp.float32),
            jax.ShapeDtypeStruct((N, 3), jnp.float32),
        ],
    )(h, x, *aggs, *xaggs, mk, w1a, w1b, b1, w2, b2)


def kernel(h, x, edge_index, mask_ligand, edge_attr,
           edge_W1, edge_b1, edge_W2, edge_b2,
           gate_W, gate_b,
           node_W1, node_b1, node_W2, node_b2,
           xm_W1, xm_b1, xm_W2):
    ei = edge_index.astype(jnp.int32)
    src = ei[0]
    dst = ei[1]

    perm = jnp.concatenate([jnp.arange(0, HID, 2), jnp.arange(1, HID, 2)])
    waP = edge_W1[:HID][:, perm]
    wbP = edge_W1[HID:2 * HID][:, perm]
    w1eP = edge_W1[2 * HID:][:, perm]
    b1P = edge_b1[perm]
    w2P = edge_W2[perm, :]

    haP, hbP, xp = _pre_call(h, x, waP, wbP)

    eap = jnp.reshape(
        jnp.transpose(jnp.reshape(edge_attr, (E // BE, 8, BE // 8, EF)),
                      (0, 2, 1, 3)),
        (E // 8, 128))

    dsts, mgs, wvs = [], [], []
    for k in range(K):
        dk = lax.slice(dst, (k * EC,), ((k + 1) * EC,))
        sk = lax.slice(src, (k * EC,), ((k + 1) * EC,))
        pab, xd, xs = _gather_call(haP, hbP, xp, dk, sk)
        eak = lax.slice(eap, (k * (EC // 8), 0), ((k + 1) * (EC // 8), 128))
        mg, wv = _edge_call(
            pab, xd, xs, eak,
            b1P.reshape(1, HID), w1eP, w2P,
            edge_b2.reshape(1, HID), gate_W, gate_b.reshape(1, 1),
            xm_W1, xm_b1.reshape(1, HID), xm_W2)
        dsts.append(dk)
        mgs.append(mg)
        wvs.append(wv)

    z128 = jnp.zeros((NPT, HID), jnp.float32)
    z16 = jnp.zeros((NPT, XP), jnp.float32)
    aggs, xaggs = [], []
    for group in ((0, 1), (2, 3), (4,)):
        a, xa = _scatter_call(
            [(mgs[k], wvs[k], dsts[k]) for k in group], z128, z16)
        aggs.append(a)
        xaggs.append(xa)

    mk = mask_ligand.astype(jnp.float32).reshape(N, 1)
    h_new, x_new = _node_call(
        h, x, aggs, xaggs, mk,
        node_W1[:HID], node_W1[HID:], node_b1.reshape(1, HID),
        node_W2, node_b2.reshape(1, HID))
    return (h_new, x_new)
